# baseline (speedup 1.0000x reference)
import functools

import jax
import jax.numpy as jnp
from jax import lax
from jax.experimental import pallas as pl
from jax.experimental.pallas import tpu as pltpu
from jax.experimental.pallas import tpu_sc as plsc

N = 50000
E = 800000
ED = 32
HID = 64
CLASSES = 2
G = 256

NC = 2
NS = 16
NW = NC * NS

RPW = 1664
MA = RPW // 128
NPAD = NW * RPW
NA = 50008

GRP = 128
GPW = 200
EPW = GPW * GRP
EPAD = EPW * NW
EG = EPAD // GRP

MD = 20
MS = 5
SB = GPW // MS
GPT2 = EG // NS
SD = GPT2 // MS

_mesh = plsc.VectorSubcoreMesh(core_axis_name="c", subcore_axis_name="s")
_sc_params = pltpu.CompilerParams(use_tc_tiling_on_sc=False)


def _zero_acc(zeros_hbm, acc_sh, sid):
    @pl.when(sid < 15)
    def _():
        pltpu.sync_copy(zeros_hbm.at[pl.ds(sid * 3128, 3128)],
                        acc_sh.at[pl.ds(sid * 3128, 3128)])

    @pl.when(sid == 15)
    def _():
        pltpu.sync_copy(zeros_hbm.at[pl.ds(15 * 3128, 3088)],
                        acc_sh.at[pl.ds(15 * 3128, 3088)])


def _store_acc(acc_sh, out_hbm, cid, sid):
    @pl.when(sid < 15)
    def _():
        pltpu.sync_copy(acc_sh.at[pl.ds(sid * 3128, 3128)],
                        out_hbm.at[cid, pl.ds(sid * 3128, 3128)])

    @pl.when(sid == 15)
    def _():
        pltpu.sync_copy(acc_sh.at[pl.ds(15 * 3128, 3088)],
                        out_hbm.at[cid, pl.ds(15 * 3128, 3088)])


@functools.partial(
    pl.kernel,
    out_type=[
        jax.ShapeDtypeStruct((NC, NPAD, ED), jnp.float32),
        jax.ShapeDtypeStruct((NC, NA, 16), jnp.float32),
    ],
    mesh=_mesh,
    scratch_types=[
        pltpu.VMEM((MA, 128), jnp.int32),
        pltpu.VMEM((RPW, ED), jnp.float32),
        pltpu.VMEM((MD, 128), jnp.int32),
        pltpu.VMEM((128, 16), jnp.float32),
        pltpu.VMEM_SHARED((NA, 16), jnp.float32),
        pltpu.SemaphoreType.DMA,
        pltpu.SemaphoreType.DMA,
    ],
    compiler_params=_sc_params,
)
def _sc_table(x2d_hbm, emb_hbm, edge3_hbm, ones_hbm, zeros16_hbm,
              tbl_hbm, cnt_hbm, idx_v, rows_v, dst_v, ones_v, cacc_sh, sem,
              sem_t):
    cid = lax.axis_index("c")
    sid = lax.axis_index("s")
    wid = sid * NC + cid

    base = wid * RPW
    pltpu.sync_copy(x2d_hbm.at[pl.ds(wid * MA, MA)], idx_v)
    descs = [pltpu.async_copy(emb_hbm.at[idx_v.at[j]],
                              rows_v.at[pl.ds(j * 128, 128)], sem_t)
             for j in range(MA)]

    _zero_acc(zeros16_hbm, cacc_sh, sid)
    pltpu.sync_copy(ones_hbm, ones_v)
    plsc.subcore_barrier()

    wg = wid * GPW

    def super_chunk(s, carry):
        pltpu.sync_copy(edge3_hbm.at[1, pl.ds(wg + s * MD, MD)], dst_v)
        ds2 = [pltpu.async_copy(ones_v, cacc_sh.at[dst_v.at[j]], sem,
                                add=True)
               for j in range(MD)]
        for d in ds2:
            d.wait()
        return carry

    lax.fori_loop(0, GPW // MD, super_chunk, 0)

    for d in descs:
        d.wait()
    pltpu.sync_copy(rows_v, tbl_hbm.at[0, pl.ds(base, RPW)])
    pltpu.sync_copy(rows_v, tbl_hbm.at[1, pl.ds(base, RPW)])

    plsc.subcore_barrier()
    _store_acc(cacc_sh, cnt_hbm, cid, sid)


def _msg_pass_body(edge3_hbm, acc_sh, sems, src_v, dst_v, rows_v, tbl_hbm,
                   wg, n_super):
    gsems = sems[:MS]
    sem_s = sems[MS]

    def super_chunk(s, carry):
        g = wg + s * MS
        pltpu.sync_copy(edge3_hbm.at[0, pl.ds(g, MS)], src_v)
        pltpu.sync_copy(edge3_hbm.at[1, pl.ds(g, MS)], dst_v)
        ds1 = [pltpu.async_copy(tbl_hbm.at[src_v.at[j]],
                                rows_v.at[pl.ds(j * 128, 128)], gsems[j])
               for j in range(MS)]
        ds2 = []
        for j in range(MS):
            ds1[j].wait()
            ds2.append(pltpu.async_copy(rows_v.at[pl.ds(j * 128, 128)],
                                        acc_sh.at[dst_v.at[j]], sem_s,
                                        add=True))
        for d in ds2:
            d.wait()
        return carry

    lax.fori_loop(0, n_super, super_chunk, 0)


@functools.partial(
    pl.kernel,
    out_type=jax.ShapeDtypeStruct((NC, NA, ED), jnp.float32),
    mesh=_mesh,
    scratch_types=[
        pltpu.VMEM((MS, 128), jnp.int32),
        pltpu.VMEM((MS, 128), jnp.int32),
        pltpu.VMEM((MS * 128, ED), jnp.float32),
        pltpu.VMEM_SHARED((NA, ED), jnp.float32),
    ] + [pltpu.SemaphoreType.DMA] * (MS + 1),
    compiler_params=_sc_params,
)
def _sc_l1(edge3_hbm, tbl_hbm, zeros_hbm, out_hbm, src_v, dst_v, rows_v,
           acc_sh, *sems):
    cid = lax.axis_index("c")
    sid = lax.axis_index("s")
    wid = sid * NC + cid

    _zero_acc(zeros_hbm, acc_sh, sid)
    plsc.subcore_barrier()
    _msg_pass_body(edge3_hbm, acc_sh, sems, src_v, dst_v, rows_v,
                   tbl_hbm.at[cid], wid * GPW, SB)
    plsc.subcore_barrier()
    _store_acc(acc_sh, out_hbm, cid, sid)


@functools.partial(
    pl.kernel,
    out_type=jax.ShapeDtypeStruct((NC, NA, ED), jnp.float32),
    mesh=_mesh,
    scratch_types=[
        pltpu.VMEM((MS, 128), jnp.int32),
        pltpu.VMEM((MS, 128), jnp.int32),
        pltpu.VMEM((MS * 128, ED), jnp.float32),
        pltpu.VMEM_SHARED((NA, ED), jnp.float32),
    ] + [pltpu.SemaphoreType.DMA] * (MS + 1),
    compiler_params=_sc_params,
)
def _sc_l2(edge3_hbm, h2lo_hbm, h2hi_hbm, zeros_hbm, out_hbm, src_v, dst_v,
           rows_v, acc_sh, *sems):
    cid = lax.axis_index("c")
    sid = lax.axis_index("s")

    _zero_acc(zeros_hbm, acc_sh, sid)
    plsc.subcore_barrier()

    wg = sid * GPT2

    @pl.when(cid == 0)
    def _():
        _msg_pass_body(edge3_hbm, acc_sh, sems, src_v, dst_v, rows_v,
                       h2lo_hbm, wg, SD)

    @pl.when(cid == 1)
    def _():
        _msg_pass_body(edge3_hbm, acc_sh, sems, src_v, dst_v, rows_v,
                       h2hi_hbm, wg, SD)

    plsc.subcore_barrier()
    _store_acc(acc_sh, out_hbm, cid, sid)


BN = 5000
NB = N // BN


def _tc_h2_body(tbl_ref, acc1_ref, cnt_ref, wl1_ref, bl1_ref, wr1_ref,
                lo_ref, hi_ref):
    agg = acc1_ref[0] + acc1_ref[1]
    cnt = cnt_ref[0, :, 0:1] + cnt_ref[1, :, 0:1]
    mean = agg / jnp.maximum(cnt, 1.0)
    h = tbl_ref[0]
    z = (jnp.dot(mean, wl1_ref[...], preferred_element_type=jnp.float32)
         + jnp.dot(h, wr1_ref[...], preferred_element_type=jnp.float32)
         + bl1_ref[...])
    h2 = jnp.maximum(z, 0.0)
    lo_ref[...] = h2[:, :32]
    hi_ref[...] = h2[:, 32:]


_tc_h2 = pl.pallas_call(
    _tc_h2_body,
    grid=(NB,),
    in_specs=[
        pl.BlockSpec((1, BN, ED), lambda i: (0, i, 0)),
        pl.BlockSpec((NC, BN, ED), lambda i: (0, i, 0)),
        pl.BlockSpec((NC, BN, 16), lambda i: (0, i, 0)),
        pl.BlockSpec((ED, HID), lambda i: (0, 0)),
        pl.BlockSpec((1, HID), lambda i: (0, 0)),
        pl.BlockSpec((ED, HID), lambda i: (0, 0)),
    ],
    out_specs=[
        pl.BlockSpec((BN, ED), lambda i: (i, 0)),
        pl.BlockSpec((BN, ED), lambda i: (i, 0)),
    ],
    out_shape=[
        jax.ShapeDtypeStruct((NA, ED), jnp.float32),
        jax.ShapeDtypeStruct((NA, ED), jnp.float32),
    ],
)


def _tc_out_body(lo_ref, hi_ref, acc2_ref, cnt_ref, batch_ref, wl2_ref,
                 bl2_ref, wr2_ref, wlin_ref, blin_ref, out_ref, pool_sc,
                 cnt_sc):
    i = pl.program_id(0)

    @pl.when(i == 0)
    def _():
        pool_sc[...] = jnp.zeros_like(pool_sc)
        cnt_sc[...] = jnp.zeros_like(cnt_sc)

    h2 = jnp.concatenate([lo_ref[...], hi_ref[...]], axis=1)
    agg2 = jnp.concatenate([acc2_ref[0], acc2_ref[1]], axis=1)
    cnt = cnt_ref[0, :, 0:1] + cnt_ref[1, :, 0:1]
    mean2 = agg2 / jnp.maximum(cnt, 1.0)
    z = (jnp.dot(mean2, wl2_ref[...], preferred_element_type=jnp.float32)
         + jnp.dot(h2, wr2_ref[...], preferred_element_type=jnp.float32)
         + bl2_ref[...])
    h3 = jnp.maximum(z, 0.0)

    b = batch_ref[0, 0, :]
    gid = lax.broadcasted_iota(jnp.int32, (BN, G), 1)
    oh = (b[:, None] == gid).astype(jnp.float32)
    pool_sc[...] += lax.dot_general(oh, h3, (((0,), (0,)), ((), ())),
                                    preferred_element_type=jnp.float32)
    cnt_sc[...] += jnp.sum(oh, axis=0, keepdims=True)

    @pl.when(i == NB - 1)
    def _():
        c = jnp.maximum(cnt_sc[...], 1.0)
        pooled = pool_sc[...] / c.reshape(G, 1)
        out_ref[...] = (jnp.dot(pooled, wlin_ref[...],
                                preferred_element_type=jnp.float32)
                        + blin_ref[...])


_tc_out = pl.pallas_call(
    _tc_out_body,
    grid=(NB,),
    in_specs=[
        pl.BlockSpec((BN, ED), lambda i: (i, 0)),
        pl.BlockSpec((BN, ED), lambda i: (i, 0)),
        pl.BlockSpec((NC, BN, ED), lambda i: (0, i, 0)),
        pl.BlockSpec((NC, BN, 16), lambda i: (0, i, 0)),
        pl.BlockSpec((1, 1, BN), lambda i: (i, 0, 0)),
        pl.BlockSpec((HID, HID), lambda i: (0, 0)),
        pl.BlockSpec((1, HID), lambda i: (0, 0)),
        pl.BlockSpec((HID, HID), lambda i: (0, 0)),
        pl.BlockSpec((HID, CLASSES), lambda i: (0, 0)),
        pl.BlockSpec((1, CLASSES), lambda i: (0, 0)),
    ],
    out_specs=pl.BlockSpec((G, CLASSES), lambda i: (0, 0)),
    out_shape=jax.ShapeDtypeStruct((G, CLASSES), jnp.float32),
    scratch_shapes=[
        pltpu.VMEM((G, HID), jnp.float32),
        pltpu.VMEM((1, G), jnp.float32),
    ],
)


def kernel(x, edge_index, batch, embed, Wl1, bl1, Wr1, Wl2, bl2, Wr2, Wlin,
           blin):
    x2d = jnp.concatenate(
        [x, jnp.zeros((NPAD - N,), jnp.int32)]).reshape(NPAD // 128, 128)
    edge3 = jnp.concatenate(
        [edge_index, jnp.full((2, EPAD - E), N, jnp.int32)],
        axis=1).reshape(2, EG, GRP)
    ones16 = jnp.ones((GRP, 16), jnp.float32)
    zeros16 = jnp.zeros((NA, 16), jnp.float32)
    zeros32 = jnp.zeros((NA, ED), jnp.float32)
    batch3 = batch.reshape(NB, 1, BN)

    tbl, cnt16 = _sc_table(x2d, embed, edge3, ones16, zeros16)
    acc1 = _sc_l1(edge3, tbl, zeros32)
    h2lo, h2hi = _tc_h2(tbl, acc1, cnt16, Wl1, bl1.reshape(1, HID), Wr1)
    acc2 = _sc_l2(edge3, h2lo, h2hi, zeros32)
    out = _tc_out(h2lo, h2hi, acc2, cnt16, batch3, Wl2, bl2.reshape(1, HID),
                  Wr2, Wlin, blin.reshape(1, CLASSES))
    return out


# speedup vs baseline: 1.0521x; 1.0521x over previous
import functools

import jax
import jax.numpy as jnp
from jax import lax
from jax.experimental import pallas as pl
from jax.experimental.pallas import tpu as pltpu
from jax.experimental.pallas import tpu_sc as plsc

N = 50000
E = 800000
ED = 32
HID = 64
CLASSES = 2
G = 256

NC = 2
NS = 16
NW = NC * NS

RPW = 1664
MA = RPW // 128
NPAD = NW * RPW
NA = 50008

GRP = 128
GPW = 200
EPW = GPW * GRP
EPAD = EPW * NW
EG = EPAD // GRP

MD = 20
MS = 5
SB = GPW // MS
GPT2 = EG // NS
SD = GPT2 // MS

_mesh = plsc.VectorSubcoreMesh(core_axis_name="c", subcore_axis_name="s")
_sc_params = pltpu.CompilerParams(use_tc_tiling_on_sc=False)


def _zero_acc(zeros_hbm, acc_sh, sid):
    @pl.when(sid < 15)
    def _():
        pltpu.sync_copy(zeros_hbm.at[pl.ds(sid * 3128, 3128)],
                        acc_sh.at[pl.ds(sid * 3128, 3128)])

    @pl.when(sid == 15)
    def _():
        pltpu.sync_copy(zeros_hbm.at[pl.ds(15 * 3128, 3088)],
                        acc_sh.at[pl.ds(15 * 3128, 3088)])


def _store_acc(acc_sh, out_hbm, cid, sid):
    @pl.when(sid < 15)
    def _():
        pltpu.sync_copy(acc_sh.at[pl.ds(sid * 3128, 3128)],
                        out_hbm.at[cid, pl.ds(sid * 3128, 3128)])

    @pl.when(sid == 15)
    def _():
        pltpu.sync_copy(acc_sh.at[pl.ds(15 * 3128, 3088)],
                        out_hbm.at[cid, pl.ds(15 * 3128, 3088)])


@functools.partial(
    pl.kernel,
    out_type=[
        jax.ShapeDtypeStruct((NPAD, ED), jnp.float32),
        jax.ShapeDtypeStruct((NC, NA, 16), jnp.float32),
    ],
    mesh=_mesh,
    scratch_types=[
        pltpu.VMEM((MA, 128), jnp.int32),
        pltpu.VMEM((RPW, ED), jnp.float32),
        pltpu.VMEM((MD, 128), jnp.int32),
        pltpu.VMEM((128, 16), jnp.float32),
        pltpu.VMEM_SHARED((NA, 16), jnp.float32),
        pltpu.SemaphoreType.DMA,
        pltpu.SemaphoreType.DMA,
    ],
    compiler_params=_sc_params,
)
def _sc_table(x2d_hbm, emb_hbm, edge3_hbm, ones_hbm, zeros16_hbm,
              tbl_hbm, cnt_hbm, idx_v, rows_v, dst_v, ones_v, cacc_sh, sem,
              sem_t):
    cid = lax.axis_index("c")
    sid = lax.axis_index("s")
    wid = sid * NC + cid

    base = wid * RPW
    pltpu.sync_copy(x2d_hbm.at[pl.ds(wid * MA, MA)], idx_v)
    descs = [pltpu.async_copy(emb_hbm.at[idx_v.at[j]],
                              rows_v.at[pl.ds(j * 128, 128)], sem_t)
             for j in range(MA)]

    _zero_acc(zeros16_hbm, cacc_sh, sid)
    pltpu.sync_copy(ones_hbm, ones_v)
    plsc.subcore_barrier()

    wg = wid * GPW

    def super_chunk(s, carry):
        pltpu.sync_copy(edge3_hbm.at[1, pl.ds(wg + s * MD, MD)], dst_v)
        ds2 = [pltpu.async_copy(ones_v, cacc_sh.at[dst_v.at[j]], sem,
                                add=True)
               for j in range(MD)]
        for d in ds2:
            d.wait()
        return carry

    lax.fori_loop(0, GPW // MD, super_chunk, 0)

    for d in descs:
        d.wait()
    pltpu.sync_copy(rows_v, tbl_hbm.at[pl.ds(base, RPW)])

    plsc.subcore_barrier()
    _store_acc(cacc_sh, cnt_hbm, cid, sid)


def _msg_pass_body(edge3_hbm, acc_sh, sems, src_v, dst_v, rows_v, tbl_hbm,
                   wg, n_super):
    gsems = sems[:MS]
    sem_s = sems[MS]

    def super_chunk(s, carry):
        g = wg + s * MS
        pltpu.sync_copy(edge3_hbm.at[0, pl.ds(g, MS)], src_v)
        pltpu.sync_copy(edge3_hbm.at[1, pl.ds(g, MS)], dst_v)
        ds1 = [pltpu.async_copy(tbl_hbm.at[src_v.at[j]],
                                rows_v.at[pl.ds(j * 128, 128)], gsems[j])
               for j in range(MS)]
        ds2 = []
        for j in range(MS):
            ds1[j].wait()
            ds2.append(pltpu.async_copy(rows_v.at[pl.ds(j * 128, 128)],
                                        acc_sh.at[dst_v.at[j]], sem_s,
                                        add=True))
        for d in ds2:
            d.wait()
        return carry

    lax.fori_loop(0, n_super, super_chunk, 0)


@functools.partial(
    pl.kernel,
    out_type=jax.ShapeDtypeStruct((NC, NA, ED), jnp.float32),
    mesh=_mesh,
    scratch_types=[
        pltpu.VMEM((MS, 128), jnp.int32),
        pltpu.VMEM((MS, 128), jnp.int32),
        pltpu.VMEM((MS * 128, ED), jnp.float32),
        pltpu.VMEM_SHARED((NA, ED), jnp.float32),
    ] + [pltpu.SemaphoreType.DMA] * (MS + 1),
    compiler_params=_sc_params,
)
def _sc_l1(edge3_hbm, tbl_hbm, zeros_hbm, out_hbm, src_v, dst_v, rows_v,
           acc_sh, *sems):
    cid = lax.axis_index("c")
    sid = lax.axis_index("s")
    wid = sid * NC + cid

    _zero_acc(zeros_hbm, acc_sh, sid)
    plsc.subcore_barrier()
    _msg_pass_body(edge3_hbm, acc_sh, sems, src_v, dst_v, rows_v, tbl_hbm,
                   wid * GPW, SB)
    plsc.subcore_barrier()
    _store_acc(acc_sh, out_hbm, cid, sid)


@functools.partial(
    pl.kernel,
    out_type=jax.ShapeDtypeStruct((NC, NA, ED), jnp.float32),
    mesh=_mesh,
    scratch_types=[
        pltpu.VMEM((MS, 128), jnp.int32),
        pltpu.VMEM((MS, 128), jnp.int32),
        pltpu.VMEM((MS * 128, ED), jnp.float32),
        pltpu.VMEM_SHARED((NA, ED), jnp.float32),
    ] + [pltpu.SemaphoreType.DMA] * (MS + 1),
    compiler_params=_sc_params,
)
def _sc_l2(edge3_hbm, h2lo_hbm, h2hi_hbm, zeros_hbm, out_hbm, src_v, dst_v,
           rows_v, acc_sh, *sems):
    cid = lax.axis_index("c")
    sid = lax.axis_index("s")

    _zero_acc(zeros_hbm, acc_sh, sid)
    plsc.subcore_barrier()

    wg = sid * GPT2

    @pl.when(cid == 0)
    def _():
        _msg_pass_body(edge3_hbm, acc_sh, sems, src_v, dst_v, rows_v,
                       h2lo_hbm, wg, SD)

    @pl.when(cid == 1)
    def _():
        _msg_pass_body(edge3_hbm, acc_sh, sems, src_v, dst_v, rows_v,
                       h2hi_hbm, wg, SD)

    plsc.subcore_barrier()
    _store_acc(acc_sh, out_hbm, cid, sid)


BN = 5000
NB = N // BN


def _tc_h2_body(tbl_ref, acc1_ref, cnt_ref, wl1_ref, bl1_ref, wr1_ref,
                lo_ref, hi_ref):
    agg = acc1_ref[0] + acc1_ref[1]
    cnt = cnt_ref[0, :, 0:1] + cnt_ref[1, :, 0:1]
    mean = agg / jnp.maximum(cnt, 1.0)
    h = tbl_ref[...]
    z = (jnp.dot(mean, wl1_ref[...], preferred_element_type=jnp.float32)
         + jnp.dot(h, wr1_ref[...], preferred_element_type=jnp.float32)
         + bl1_ref[...])
    h2 = jnp.maximum(z, 0.0)
    lo_ref[...] = h2[:, :32]
    hi_ref[...] = h2[:, 32:]


_tc_h2 = pl.pallas_call(
    _tc_h2_body,
    grid=(NB,),
    in_specs=[
        pl.BlockSpec((BN, ED), lambda i: (i, 0)),
        pl.BlockSpec((NC, BN, ED), lambda i: (0, i, 0)),
        pl.BlockSpec((NC, BN, 16), lambda i: (0, i, 0)),
        pl.BlockSpec((ED, HID), lambda i: (0, 0)),
        pl.BlockSpec((1, HID), lambda i: (0, 0)),
        pl.BlockSpec((ED, HID), lambda i: (0, 0)),
    ],
    out_specs=[
        pl.BlockSpec((BN, ED), lambda i: (i, 0)),
        pl.BlockSpec((BN, ED), lambda i: (i, 0)),
    ],
    out_shape=[
        jax.ShapeDtypeStruct((NA, ED), jnp.float32),
        jax.ShapeDtypeStruct((NA, ED), jnp.float32),
    ],
)


def _tc_out_body(lo_ref, hi_ref, acc2_ref, cnt_ref, batch_ref, wl2_ref,
                 bl2_ref, wr2_ref, wlin_ref, blin_ref, out_ref, pool_sc,
                 cnt_sc):
    i = pl.program_id(0)

    @pl.when(i == 0)
    def _():
        pool_sc[...] = jnp.zeros_like(pool_sc)
        cnt_sc[...] = jnp.zeros_like(cnt_sc)

    h2 = jnp.concatenate([lo_ref[...], hi_ref[...]], axis=1)
    agg2 = jnp.concatenate([acc2_ref[0], acc2_ref[1]], axis=1)
    cnt = cnt_ref[0, :, 0:1] + cnt_ref[1, :, 0:1]
    mean2 = agg2 / jnp.maximum(cnt, 1.0)
    z = (jnp.dot(mean2, wl2_ref[...], preferred_element_type=jnp.float32)
         + jnp.dot(h2, wr2_ref[...], preferred_element_type=jnp.float32)
         + bl2_ref[...])
    h3 = jnp.maximum(z, 0.0)

    b = batch_ref[0, 0, :]
    gid = lax.broadcasted_iota(jnp.int32, (BN, G), 1)
    oh = (b[:, None] == gid).astype(jnp.float32)
    pool_sc[...] += lax.dot_general(oh, h3, (((0,), (0,)), ((), ())),
                                    preferred_element_type=jnp.float32)
    cnt_sc[...] += jnp.sum(oh, axis=0, keepdims=True)

    @pl.when(i == NB - 1)
    def _():
        c = jnp.maximum(cnt_sc[...], 1.0)
        pooled = pool_sc[...] / c.reshape(G, 1)
        out_ref[...] = (jnp.dot(pooled, wlin_ref[...],
                                preferred_element_type=jnp.float32)
                        + blin_ref[...])


_tc_out = pl.pallas_call(
    _tc_out_body,
    grid=(NB,),
    in_specs=[
        pl.BlockSpec((BN, ED), lambda i: (i, 0)),
        pl.BlockSpec((BN, ED), lambda i: (i, 0)),
        pl.BlockSpec((NC, BN, ED), lambda i: (0, i, 0)),
        pl.BlockSpec((NC, BN, 16), lambda i: (0, i, 0)),
        pl.BlockSpec((1, 1, BN), lambda i: (i, 0, 0)),
        pl.BlockSpec((HID, HID), lambda i: (0, 0)),
        pl.BlockSpec((1, HID), lambda i: (0, 0)),
        pl.BlockSpec((HID, HID), lambda i: (0, 0)),
        pl.BlockSpec((HID, CLASSES), lambda i: (0, 0)),
        pl.BlockSpec((1, CLASSES), lambda i: (0, 0)),
    ],
    out_specs=pl.BlockSpec((G, CLASSES), lambda i: (0, 0)),
    out_shape=jax.ShapeDtypeStruct((G, CLASSES), jnp.float32),
    scratch_shapes=[
        pltpu.VMEM((G, HID), jnp.float32),
        pltpu.VMEM((1, G), jnp.float32),
    ],
)


def kernel(x, edge_index, batch, embed, Wl1, bl1, Wr1, Wl2, bl2, Wr2, Wlin,
           blin):
    x2d = jnp.concatenate(
        [x, jnp.zeros((NPAD - N,), jnp.int32)]).reshape(NPAD // 128, 128)
    edge3 = jnp.concatenate(
        [edge_index, jnp.full((2, EPAD - E), N, jnp.int32)],
        axis=1).reshape(2, EG, GRP)
    ones16 = jnp.ones((GRP, 16), jnp.float32)
    zeros16 = jnp.zeros((NA, 16), jnp.float32)
    zeros32 = jnp.zeros((NA, ED), jnp.float32)
    batch3 = batch.reshape(NB, 1, BN)

    tbl, cnt16 = _sc_table(x2d, embed, edge3, ones16, zeros16)
    acc1 = _sc_l1(edge3, tbl, zeros32)
    h2lo, h2hi = _tc_h2(tbl, acc1, cnt16, Wl1, bl1.reshape(1, HID), Wr1)
    acc2 = _sc_l2(edge3, h2lo, h2hi, zeros32)
    out = _tc_out(h2lo, h2hi, acc2, cnt16, batch3, Wl2, bl2.reshape(1, HID),
                  Wr2, Wlin, blin.reshape(1, CLASSES))
    return out


# speedup vs baseline: 1.1246x; 1.0689x over previous
import functools

import jax
import jax.numpy as jnp
from jax import lax
from jax.experimental import pallas as pl
from jax.experimental.pallas import tpu as pltpu
from jax.experimental.pallas import tpu_sc as plsc

N = 50000
E = 800000
ED = 32
HID = 64
CLASSES = 2
G = 256

NC = 2
NS = 16
NW = NC * NS

RPW = 1664
MA = RPW // 128
NPAD = NW * RPW
NA = 50008

GRP = 128
GPW = 200
EPW = GPW * GRP
EPAD = EPW * NW
EG = EPAD // GRP

MD = 20
MS = 5
KC = 4
SB = GPW // MS
GPT2 = EG // NS
SD = GPT2 // MS

_mesh = plsc.VectorSubcoreMesh(core_axis_name="c", subcore_axis_name="s")
_sc_params = pltpu.CompilerParams(use_tc_tiling_on_sc=False)


def _zero_acc(zeros_hbm, acc_sh, sid):
    @pl.when(sid < 15)
    def _():
        pltpu.sync_copy(zeros_hbm.at[pl.ds(sid * 3128, 3128)],
                        acc_sh.at[pl.ds(sid * 3128, 3128)])

    @pl.when(sid == 15)
    def _():
        pltpu.sync_copy(zeros_hbm.at[pl.ds(15 * 3128, 3088)],
                        acc_sh.at[pl.ds(15 * 3128, 3088)])


def _store_acc(acc_sh, out_hbm, cid, sid):
    @pl.when(sid < 15)
    def _():
        pltpu.sync_copy(acc_sh.at[pl.ds(sid * 3128, 3128)],
                        out_hbm.at[cid, pl.ds(sid * 3128, 3128)])

    @pl.when(sid == 15)
    def _():
        pltpu.sync_copy(acc_sh.at[pl.ds(15 * 3128, 3088)],
                        out_hbm.at[cid, pl.ds(15 * 3128, 3088)])


@functools.partial(
    pl.kernel,
    out_type=[
        jax.ShapeDtypeStruct((NPAD, ED), jnp.float32),
        jax.ShapeDtypeStruct((NC, NA, 16), jnp.float32),
    ],
    mesh=_mesh,
    scratch_types=[
        pltpu.VMEM((MA, 128), jnp.int32),
        pltpu.VMEM((RPW, ED), jnp.float32),
        pltpu.VMEM((MD, 128), jnp.int32),
        pltpu.VMEM((128, 16), jnp.float32),
        pltpu.VMEM_SHARED((NA, 16), jnp.float32),
        pltpu.SemaphoreType.DMA,
        pltpu.SemaphoreType.DMA,
    ],
    compiler_params=_sc_params,
)
def _sc_table(x2d_hbm, emb_hbm, edge3_hbm, ones_hbm, zeros16_hbm,
              tbl_hbm, cnt_hbm, idx_v, rows_v, dst_v, ones_v, cacc_sh, sem,
              sem_t):
    cid = lax.axis_index("c")
    sid = lax.axis_index("s")
    wid = sid * NC + cid

    base = wid * RPW
    pltpu.sync_copy(x2d_hbm.at[pl.ds(wid * MA, MA)], idx_v)
    descs = [pltpu.async_copy(emb_hbm.at[idx_v.at[j]],
                              rows_v.at[pl.ds(j * 128, 128)], sem_t)
             for j in range(MA)]

    _zero_acc(zeros16_hbm, cacc_sh, sid)
    pltpu.sync_copy(ones_hbm, ones_v)
    plsc.subcore_barrier()

    wg = wid * GPW

    def super_chunk(s, carry):
        pltpu.sync_copy(edge3_hbm.at[1, pl.ds(wg + s * MD, MD)], dst_v)
        ds2 = [pltpu.async_copy(ones_v, cacc_sh.at[dst_v.at[j]], sem,
                                add=True)
               for j in range(MD)]
        for d in ds2:
            d.wait()
        return carry

    lax.fori_loop(0, GPW // MD, super_chunk, 0)

    for d in descs:
        d.wait()
    pltpu.sync_copy(rows_v, tbl_hbm.at[pl.ds(base, RPW)])

    plsc.subcore_barrier()
    _store_acc(cacc_sh, cnt_hbm, cid, sid)


def _msg_pass_body(edge3_hbm, acc_sh, sems, src_v, dst_v, rows_v, tbl_hbm,
                   wg, n_super):
    gsems = sems[:MS]
    sem_s = sems[MS]

    def super_chunk(s, carry):
        g = wg + s * (KC * MS)
        pltpu.sync_copy(edge3_hbm.at[0, pl.ds(g, KC * MS)], src_v)
        pltpu.sync_copy(edge3_hbm.at[1, pl.ds(g, KC * MS)], dst_v)
        for k in range(KC):
            ds1 = [pltpu.async_copy(tbl_hbm.at[src_v.at[k * MS + j]],
                                    rows_v.at[pl.ds(j * 128, 128)], gsems[j])
                   for j in range(MS)]
            ds2 = []
            for j in range(MS):
                ds1[j].wait()
                ds2.append(pltpu.async_copy(rows_v.at[pl.ds(j * 128, 128)],
                                            acc_sh.at[dst_v.at[k * MS + j]],
                                            sem_s, add=True))
            for d in ds2:
                d.wait()
        return carry

    lax.fori_loop(0, n_super // KC, super_chunk, 0)


@functools.partial(
    pl.kernel,
    out_type=jax.ShapeDtypeStruct((NC, NA, ED), jnp.float32),
    mesh=_mesh,
    scratch_types=[
        pltpu.VMEM((KC * MS, 128), jnp.int32),
        pltpu.VMEM((KC * MS, 128), jnp.int32),
        pltpu.VMEM((MS * 128, ED), jnp.float32),
        pltpu.VMEM_SHARED((NA, ED), jnp.float32),
    ] + [pltpu.SemaphoreType.DMA] * (MS + 1),
    compiler_params=_sc_params,
)
def _sc_l1(edge3_hbm, tbl_hbm, zeros_hbm, out_hbm, src_v, dst_v, rows_v,
           acc_sh, *sems):
    cid = lax.axis_index("c")
    sid = lax.axis_index("s")
    wid = sid * NC + cid

    _zero_acc(zeros_hbm, acc_sh, sid)
    plsc.subcore_barrier()
    _msg_pass_body(edge3_hbm, acc_sh, sems, src_v, dst_v, rows_v, tbl_hbm,
                   wid * GPW, SB)
    plsc.subcore_barrier()
    _store_acc(acc_sh, out_hbm, cid, sid)


@functools.partial(
    pl.kernel,
    out_type=jax.ShapeDtypeStruct((NC, NA, ED), jnp.float32),
    mesh=_mesh,
    scratch_types=[
        pltpu.VMEM((KC * MS, 128), jnp.int32),
        pltpu.VMEM((KC * MS, 128), jnp.int32),
        pltpu.VMEM((MS * 128, ED), jnp.float32),
        pltpu.VMEM_SHARED((NA, ED), jnp.float32),
    ] + [pltpu.SemaphoreType.DMA] * (MS + 1),
    compiler_params=_sc_params,
)
def _sc_l2(edge3_hbm, h2lo_hbm, h2hi_hbm, zeros_hbm, out_hbm, src_v, dst_v,
           rows_v, acc_sh, *sems):
    cid = lax.axis_index("c")
    sid = lax.axis_index("s")

    _zero_acc(zeros_hbm, acc_sh, sid)
    plsc.subcore_barrier()

    wg = sid * GPT2

    @pl.when(cid == 0)
    def _():
        _msg_pass_body(edge3_hbm, acc_sh, sems, src_v, dst_v, rows_v,
                       h2lo_hbm, wg, SD)

    @pl.when(cid == 1)
    def _():
        _msg_pass_body(edge3_hbm, acc_sh, sems, src_v, dst_v, rows_v,
                       h2hi_hbm, wg, SD)

    plsc.subcore_barrier()
    _store_acc(acc_sh, out_hbm, cid, sid)


BN = 5000
NB = N // BN


def _tc_h2_body(tbl_ref, acc1_ref, cnt_ref, wl1_ref, bl1_ref, wr1_ref,
                lo_ref, hi_ref):
    agg = acc1_ref[0] + acc1_ref[1]
    cnt = cnt_ref[0, :, 0:1] + cnt_ref[1, :, 0:1]
    mean = agg / jnp.maximum(cnt, 1.0)
    h = tbl_ref[...]
    z = (jnp.dot(mean, wl1_ref[...], preferred_element_type=jnp.float32)
         + jnp.dot(h, wr1_ref[...], preferred_element_type=jnp.float32)
         + bl1_ref[...])
    h2 = jnp.maximum(z, 0.0)
    lo_ref[...] = h2[:, :32]
    hi_ref[...] = h2[:, 32:]


_tc_h2 = pl.pallas_call(
    _tc_h2_body,
    grid=(NB,),
    in_specs=[
        pl.BlockSpec((BN, ED), lambda i: (i, 0)),
        pl.BlockSpec((NC, BN, ED), lambda i: (0, i, 0)),
        pl.BlockSpec((NC, BN, 16), lambda i: (0, i, 0)),
        pl.BlockSpec((ED, HID), lambda i: (0, 0)),
        pl.BlockSpec((1, HID), lambda i: (0, 0)),
        pl.BlockSpec((ED, HID), lambda i: (0, 0)),
    ],
    out_specs=[
        pl.BlockSpec((BN, ED), lambda i: (i, 0)),
        pl.BlockSpec((BN, ED), lambda i: (i, 0)),
    ],
    out_shape=[
        jax.ShapeDtypeStruct((NA, ED), jnp.float32),
        jax.ShapeDtypeStruct((NA, ED), jnp.float32),
    ],
)


def _tc_out_body(lo_ref, hi_ref, acc2_ref, cnt_ref, batch_ref, wl2_ref,
                 bl2_ref, wr2_ref, wlin_ref, blin_ref, out_ref, pool_sc,
                 cnt_sc):
    i = pl.program_id(0)

    @pl.when(i == 0)
    def _():
        pool_sc[...] = jnp.zeros_like(pool_sc)
        cnt_sc[...] = jnp.zeros_like(cnt_sc)

    h2 = jnp.concatenate([lo_ref[...], hi_ref[...]], axis=1)
    agg2 = jnp.concatenate([acc2_ref[0], acc2_ref[1]], axis=1)
    cnt = cnt_ref[0, :, 0:1] + cnt_ref[1, :, 0:1]
    mean2 = agg2 / jnp.maximum(cnt, 1.0)
    z = (jnp.dot(mean2, wl2_ref[...], preferred_element_type=jnp.float32)
         + jnp.dot(h2, wr2_ref[...], preferred_element_type=jnp.float32)
         + bl2_ref[...])
    h3 = jnp.maximum(z, 0.0)

    b = batch_ref[0, 0, :]
    gid = lax.broadcasted_iota(jnp.int32, (BN, G), 1)
    oh = (b[:, None] == gid).astype(jnp.float32)
    pool_sc[...] += lax.dot_general(oh, h3, (((0,), (0,)), ((), ())),
                                    preferred_element_type=jnp.float32)
    cnt_sc[...] += jnp.sum(oh, axis=0, keepdims=True)

    @pl.when(i == NB - 1)
    def _():
        c = jnp.maximum(cnt_sc[...], 1.0)
        pooled = pool_sc[...] / c.reshape(G, 1)
        out_ref[...] = (jnp.dot(pooled, wlin_ref[...],
                                preferred_element_type=jnp.float32)
                        + blin_ref[...])


_tc_out = pl.pallas_call(
    _tc_out_body,
    grid=(NB,),
    in_specs=[
        pl.BlockSpec((BN, ED), lambda i: (i, 0)),
        pl.BlockSpec((BN, ED), lambda i: (i, 0)),
        pl.BlockSpec((NC, BN, ED), lambda i: (0, i, 0)),
        pl.BlockSpec((NC, BN, 16), lambda i: (0, i, 0)),
        pl.BlockSpec((1, 1, BN), lambda i: (i, 0, 0)),
        pl.BlockSpec((HID, HID), lambda i: (0, 0)),
        pl.BlockSpec((1, HID), lambda i: (0, 0)),
        pl.BlockSpec((HID, HID), lambda i: (0, 0)),
        pl.BlockSpec((HID, CLASSES), lambda i: (0, 0)),
        pl.BlockSpec((1, CLASSES), lambda i: (0, 0)),
    ],
    out_specs=pl.BlockSpec((G, CLASSES), lambda i: (0, 0)),
    out_shape=jax.ShapeDtypeStruct((G, CLASSES), jnp.float32),
    scratch_shapes=[
        pltpu.VMEM((G, HID), jnp.float32),
        pltpu.VMEM((1, G), jnp.float32),
    ],
)


def kernel(x, edge_index, batch, embed, Wl1, bl1, Wr1, Wl2, bl2, Wr2, Wlin,
           blin):
    x2d = jnp.concatenate(
        [x, jnp.zeros((NPAD - N,), jnp.int32)]).reshape(NPAD // 128, 128)
    edge3 = jnp.concatenate(
        [edge_index, jnp.full((2, EPAD - E), N, jnp.int32)],
        axis=1).reshape(2, EG, GRP)
    ones16 = jnp.ones((GRP, 16), jnp.float32)
    zeros16 = jnp.zeros((NA, 16), jnp.float32)
    zeros32 = jnp.zeros((NA, ED), jnp.float32)
    batch3 = batch.reshape(NB, 1, BN)

    tbl, cnt16 = _sc_table(x2d, embed, edge3, ones16, zeros16)
    acc1 = _sc_l1(edge3, tbl, zeros32)
    h2lo, h2hi = _tc_h2(tbl, acc1, cnt16, Wl1, bl1.reshape(1, HID), Wr1)
    acc2 = _sc_l2(edge3, h2lo, h2hi, zeros32)
    out = _tc_out(h2lo, h2hi, acc2, cnt16, batch3, Wl2, bl2.reshape(1, HID),
                  Wr2, Wlin, blin.reshape(1, CLASSES))
    return out


# speedup vs baseline: 1.1383x; 1.0122x over previous
import functools

import jax
import jax.numpy as jnp
from jax import lax
from jax.experimental import pallas as pl
from jax.experimental.pallas import tpu as pltpu
from jax.experimental.pallas import tpu_sc as plsc

N = 50000
E = 800000
ED = 32
HID = 64
CLASSES = 2
G = 256

NC = 2
NS = 16
NW = NC * NS

RPW = 1664
MA = RPW // 128
NPAD = NW * RPW
NA = 50008

GRP = 128
GPW = 200
EPW = GPW * GRP
EPAD = EPW * NW
EG = EPAD // GRP

MD = 20
MS = 5
KC = 8
SB = GPW // MS
GPT2 = EG // NS
SD = GPT2 // MS

_mesh = plsc.VectorSubcoreMesh(core_axis_name="c", subcore_axis_name="s")
_sc_params = pltpu.CompilerParams(use_tc_tiling_on_sc=False)


def _zero_acc(zeros_hbm, acc_sh, sid):
    @pl.when(sid < 15)
    def _():
        pltpu.sync_copy(zeros_hbm.at[pl.ds(sid * 3128, 3128)],
                        acc_sh.at[pl.ds(sid * 3128, 3128)])

    @pl.when(sid == 15)
    def _():
        pltpu.sync_copy(zeros_hbm.at[pl.ds(15 * 3128, 3088)],
                        acc_sh.at[pl.ds(15 * 3128, 3088)])


def _store_acc(acc_sh, out_hbm, cid, sid):
    @pl.when(sid < 15)
    def _():
        pltpu.sync_copy(acc_sh.at[pl.ds(sid * 3128, 3128)],
                        out_hbm.at[cid, pl.ds(sid * 3128, 3128)])

    @pl.when(sid == 15)
    def _():
        pltpu.sync_copy(acc_sh.at[pl.ds(15 * 3128, 3088)],
                        out_hbm.at[cid, pl.ds(15 * 3128, 3088)])


@functools.partial(
    pl.kernel,
    out_type=[
        jax.ShapeDtypeStruct((NPAD, ED), jnp.float32),
        jax.ShapeDtypeStruct((NC, NA, 16), jnp.float32),
    ],
    mesh=_mesh,
    scratch_types=[
        pltpu.VMEM((MA, 128), jnp.int32),
        pltpu.VMEM((RPW, ED), jnp.float32),
        pltpu.VMEM((MD, 128), jnp.int32),
        pltpu.VMEM((128, 16), jnp.float32),
        pltpu.VMEM_SHARED((NA, 16), jnp.float32),
        pltpu.SemaphoreType.DMA,
        pltpu.SemaphoreType.DMA,
    ],
    compiler_params=_sc_params,
)
def _sc_table(x2d_hbm, emb_hbm, edge3_hbm, ones_hbm, zeros16_hbm,
              tbl_hbm, cnt_hbm, idx_v, rows_v, dst_v, ones_v, cacc_sh, sem,
              sem_t):
    cid = lax.axis_index("c")
    sid = lax.axis_index("s")
    wid = sid * NC + cid

    base = wid * RPW
    pltpu.sync_copy(x2d_hbm.at[pl.ds(wid * MA, MA)], idx_v)
    descs = [pltpu.async_copy(emb_hbm.at[idx_v.at[j]],
                              rows_v.at[pl.ds(j * 128, 128)], sem_t)
             for j in range(MA)]

    _zero_acc(zeros16_hbm, cacc_sh, sid)
    pltpu.sync_copy(ones_hbm, ones_v)
    plsc.subcore_barrier()

    wg = wid * GPW

    def super_chunk(s, carry):
        pltpu.sync_copy(edge3_hbm.at[1, pl.ds(wg + s * MD, MD)], dst_v)
        ds2 = [pltpu.async_copy(ones_v, cacc_sh.at[dst_v.at[j]], sem,
                                add=True)
               for j in range(MD)]
        for d in ds2:
            d.wait()
        return carry

    lax.fori_loop(0, GPW // MD, super_chunk, 0)

    for d in descs:
        d.wait()
    pltpu.sync_copy(rows_v, tbl_hbm.at[pl.ds(base, RPW)])

    plsc.subcore_barrier()
    _store_acc(cacc_sh, cnt_hbm, cid, sid)


def _msg_pass_body(edge3_hbm, acc_sh, sems, src_v, dst_v, rows_v, tbl_hbm,
                   wg, n_super):
    gsems = sems[:MS]
    sem_s = sems[MS]

    def super_chunk(s, carry):
        g = wg + s * (KC * MS)
        pltpu.sync_copy(edge3_hbm.at[0, pl.ds(g, KC * MS)], src_v)
        pltpu.sync_copy(edge3_hbm.at[1, pl.ds(g, KC * MS)], dst_v)
        for k in range(KC):
            ds1 = [pltpu.async_copy(tbl_hbm.at[src_v.at[k * MS + j]],
                                    rows_v.at[pl.ds(j * 128, 128)], gsems[j])
                   for j in range(MS)]
            ds2 = []
            for j in range(MS):
                ds1[j].wait()
                ds2.append(pltpu.async_copy(rows_v.at[pl.ds(j * 128, 128)],
                                            acc_sh.at[dst_v.at[k * MS + j]],
                                            sem_s, add=True))
            for d in ds2:
                d.wait()
        return carry

    lax.fori_loop(0, n_super // KC, super_chunk, 0)


@functools.partial(
    pl.kernel,
    out_type=jax.ShapeDtypeStruct((NC, NA, ED), jnp.float32),
    mesh=_mesh,
    scratch_types=[
        pltpu.VMEM((KC * MS, 128), jnp.int32),
        pltpu.VMEM((KC * MS, 128), jnp.int32),
        pltpu.VMEM((MS * 128, ED), jnp.float32),
        pltpu.VMEM_SHARED((NA, ED), jnp.float32),
    ] + [pltpu.SemaphoreType.DMA] * (MS + 1),
    compiler_params=_sc_params,
)
def _sc_l1(edge3_hbm, tbl_hbm, zeros_hbm, out_hbm, src_v, dst_v, rows_v,
           acc_sh, *sems):
    cid = lax.axis_index("c")
    sid = lax.axis_index("s")
    wid = sid * NC + cid

    _zero_acc(zeros_hbm, acc_sh, sid)
    plsc.subcore_barrier()
    _msg_pass_body(edge3_hbm, acc_sh, sems, src_v, dst_v, rows_v, tbl_hbm,
                   wid * GPW, SB)
    plsc.subcore_barrier()
    _store_acc(acc_sh, out_hbm, cid, sid)


@functools.partial(
    pl.kernel,
    out_type=jax.ShapeDtypeStruct((NC, NA, ED), jnp.float32),
    mesh=_mesh,
    scratch_types=[
        pltpu.VMEM((KC * MS, 128), jnp.int32),
        pltpu.VMEM((KC * MS, 128), jnp.int32),
        pltpu.VMEM((MS * 128, ED), jnp.float32),
        pltpu.VMEM_SHARED((NA, ED), jnp.float32),
    ] + [pltpu.SemaphoreType.DMA] * (MS + 1),
    compiler_params=_sc_params,
)
def _sc_l2(edge3_hbm, h2lo_hbm, h2hi_hbm, zeros_hbm, out_hbm, src_v, dst_v,
           rows_v, acc_sh, *sems):
    cid = lax.axis_index("c")
    sid = lax.axis_index("s")

    _zero_acc(zeros_hbm, acc_sh, sid)
    plsc.subcore_barrier()

    wg = sid * GPT2

    @pl.when(cid == 0)
    def _():
        _msg_pass_body(edge3_hbm, acc_sh, sems, src_v, dst_v, rows_v,
                       h2lo_hbm, wg, SD)

    @pl.when(cid == 1)
    def _():
        _msg_pass_body(edge3_hbm, acc_sh, sems, src_v, dst_v, rows_v,
                       h2hi_hbm, wg, SD)

    plsc.subcore_barrier()
    _store_acc(acc_sh, out_hbm, cid, sid)


BN = 5000
NB = N // BN


def _tc_h2_body(tbl_ref, acc1_ref, cnt_ref, wl1_ref, bl1_ref, wr1_ref,
                lo_ref, hi_ref):
    agg = acc1_ref[0] + acc1_ref[1]
    cnt = cnt_ref[0, :, 0:1] + cnt_ref[1, :, 0:1]
    mean = agg / jnp.maximum(cnt, 1.0)
    h = tbl_ref[...]
    z = (jnp.dot(mean, wl1_ref[...], preferred_element_type=jnp.float32)
         + jnp.dot(h, wr1_ref[...], preferred_element_type=jnp.float32)
         + bl1_ref[...])
    h2 = jnp.maximum(z, 0.0)
    lo_ref[...] = h2[:, :32]
    hi_ref[...] = h2[:, 32:]


_tc_h2 = pl.pallas_call(
    _tc_h2_body,
    grid=(NB,),
    in_specs=[
        pl.BlockSpec((BN, ED), lambda i: (i, 0)),
        pl.BlockSpec((NC, BN, ED), lambda i: (0, i, 0)),
        pl.BlockSpec((NC, BN, 16), lambda i: (0, i, 0)),
        pl.BlockSpec((ED, HID), lambda i: (0, 0)),
        pl.BlockSpec((1, HID), lambda i: (0, 0)),
        pl.BlockSpec((ED, HID), lambda i: (0, 0)),
    ],
    out_specs=[
        pl.BlockSpec((BN, ED), lambda i: (i, 0)),
        pl.BlockSpec((BN, ED), lambda i: (i, 0)),
    ],
    out_shape=[
        jax.ShapeDtypeStruct((NA, ED), jnp.float32),
        jax.ShapeDtypeStruct((NA, ED), jnp.float32),
    ],
)


def _tc_out_body(lo_ref, hi_ref, acc2_ref, cnt_ref, batch_ref, wl2_ref,
                 bl2_ref, wr2_ref, wlin_ref, blin_ref, out_ref, pool_sc,
                 cnt_sc):
    i = pl.program_id(0)

    @pl.when(i == 0)
    def _():
        pool_sc[...] = jnp.zeros_like(pool_sc)
        cnt_sc[...] = jnp.zeros_like(cnt_sc)

    h2 = jnp.concatenate([lo_ref[...], hi_ref[...]], axis=1)
    agg2 = jnp.concatenate([acc2_ref[0], acc2_ref[1]], axis=1)
    cnt = cnt_ref[0, :, 0:1] + cnt_ref[1, :, 0:1]
    mean2 = agg2 / jnp.maximum(cnt, 1.0)
    z = (jnp.dot(mean2, wl2_ref[...], preferred_element_type=jnp.float32)
         + jnp.dot(h2, wr2_ref[...], preferred_element_type=jnp.float32)
         + bl2_ref[...])
    h3 = jnp.maximum(z, 0.0)

    b = batch_ref[0, 0, :]
    gid = lax.broadcasted_iota(jnp.int32, (BN, G), 1)
    oh = (b[:, None] == gid).astype(jnp.float32)
    pool_sc[...] += lax.dot_general(oh, h3, (((0,), (0,)), ((), ())),
                                    preferred_element_type=jnp.float32)
    cnt_sc[...] += jnp.sum(oh, axis=0, keepdims=True)

    @pl.when(i == NB - 1)
    def _():
        c = jnp.maximum(cnt_sc[...], 1.0)
        pooled = pool_sc[...] / c.reshape(G, 1)
        out_ref[...] = (jnp.dot(pooled, wlin_ref[...],
                                preferred_element_type=jnp.float32)
                        + blin_ref[...])


_tc_out = pl.pallas_call(
    _tc_out_body,
    grid=(NB,),
    in_specs=[
        pl.BlockSpec((BN, ED), lambda i: (i, 0)),
        pl.BlockSpec((BN, ED), lambda i: (i, 0)),
        pl.BlockSpec((NC, BN, ED), lambda i: (0, i, 0)),
        pl.BlockSpec((NC, BN, 16), lambda i: (0, i, 0)),
        pl.BlockSpec((1, 1, BN), lambda i: (i, 0, 0)),
        pl.BlockSpec((HID, HID), lambda i: (0, 0)),
        pl.BlockSpec((1, HID), lambda i: (0, 0)),
        pl.BlockSpec((HID, HID), lambda i: (0, 0)),
        pl.BlockSpec((HID, CLASSES), lambda i: (0, 0)),
        pl.BlockSpec((1, CLASSES), lambda i: (0, 0)),
    ],
    out_specs=pl.BlockSpec((G, CLASSES), lambda i: (0, 0)),
    out_shape=jax.ShapeDtypeStruct((G, CLASSES), jnp.float32),
    scratch_shapes=[
        pltpu.VMEM((G, HID), jnp.float32),
        pltpu.VMEM((1, G), jnp.float32),
    ],
)


def kernel(x, edge_index, batch, embed, Wl1, bl1, Wr1, Wl2, bl2, Wr2, Wlin,
           blin):
    x2d = jnp.concatenate(
        [x, jnp.zeros((NPAD - N,), jnp.int32)]).reshape(NPAD // 128, 128)
    edge3 = jnp.concatenate(
        [edge_index, jnp.full((2, EPAD - E), N, jnp.int32)],
        axis=1).reshape(2, EG, GRP)
    ones16 = jnp.ones((GRP, 16), jnp.float32)
    zeros16 = jnp.zeros((NA, 16), jnp.float32)
    zeros32 = jnp.zeros((NA, ED), jnp.float32)
    batch3 = batch.reshape(NB, 1, BN)

    tbl, cnt16 = _sc_table(x2d, embed, edge3, ones16, zeros16)
    acc1 = _sc_l1(edge3, tbl, zeros32)
    h2lo, h2hi = _tc_h2(tbl, acc1, cnt16, Wl1, bl1.reshape(1, HID), Wr1)
    acc2 = _sc_l2(edge3, h2lo, h2hi, zeros32)
    out = _tc_out(h2lo, h2hi, acc2, cnt16, batch3, Wl2, bl2.reshape(1, HID),
                  Wr2, Wlin, blin.reshape(1, CLASSES))
    return out
